# baseline (speedup 1.0000x reference)
import functools

import jax
import jax.numpy as jnp
from jax import lax
from jax.experimental import pallas as pl
from jax.experimental.pallas import tpu as pltpu
from jax.experimental.pallas import tpu_sc as plsc

B = 4096
F = 26
EMB = 16
TOTAL = 2600000
NC = 2
NS = 16
NW = NC * NS
BPW = B // NW
RPW = BPW * F
CH = 128
NCH = RPW // CH
HALF = NCH // 2
RH = HALF * CH
SPH = RH // F
LANES = 16

BT = 2048
GRID_T = (TOTAL + BT - 1) // BT

_mesh = plsc.VectorSubcoreMesh(core_axis_name="c", subcore_axis_name="s")


def _tp_body(vt_ref, out_ref):
    eye = jnp.float32(1.0) * (lax.broadcasted_iota(jnp.int32, (EMB, EMB), 0)
                              == lax.broadcasted_iota(jnp.int32, (EMB, EMB), 1))
    out_ref[...] = lax.dot_general(
        vt_ref[...], eye, (((0,), (0,)), ((), ())),
        preferred_element_type=jnp.float32)


_transpose = pl.pallas_call(
    _tp_body,
    grid=(GRID_T,),
    in_specs=[pl.BlockSpec((EMB, BT), lambda g: (0, g))],
    out_specs=pl.BlockSpec((BT, EMB), lambda g: (g, 0)),
    out_shape=jax.ShapeDtypeStruct((TOTAL, EMB), jnp.float32),
)


@functools.partial(
    pl.kernel,
    out_type=jax.ShapeDtypeStruct((B,), jnp.float32),
    mesh=_mesh,
    compiler_params=pltpu.CompilerParams(
        needs_layout_passes=False, use_tc_tiling_on_sc=False),
    scratch_types=[
        pltpu.VMEM((RPW,), jnp.int32),
        pltpu.VMEM((RPW, EMB), jnp.float32),
        pltpu.VMEM((RPW,), jnp.float32),
        pltpu.VMEM((BPW, LANES), jnp.float32),
        pltpu.VMEM((BPW,), jnp.float32),
        pltpu.SemaphoreType.DMA,
        pltpu.SemaphoreType.DMA,
        pltpu.SemaphoreType.DMA,
        pltpu.SemaphoreType.DMA,
    ],
)
def _fm_sc(x_hbm, w_hbm, v_hbm, out_hbm,
           idx_v, rows_v, w_v, part_v, out_v, sem_v0, sem_v1, sem_w0, sem_w1):
    wid = lax.axis_index("s") * NC + lax.axis_index("c")
    base = wid * RPW

    pltpu.sync_copy(x_hbm.at[pl.ds(base, RPW)], idx_v)

    def fire(j, sem_v, sem_w):
        off = j * CH
        idx_c = idx_v.at[pl.ds(off, CH)]
        pltpu.async_copy(v_hbm.at[idx_c], rows_v.at[pl.ds(off, CH)], sem_v)
        pltpu.async_copy(w_hbm.at[idx_c], w_v.at[pl.ds(off, CH)], sem_w)

    lax.fori_loop(0, HALF, lambda j, _: (fire(j, sem_v0, sem_w0), 0)[1], 0)
    lax.fori_loop(HALF, NCH, lambda j, _: (fire(j, sem_v1, sem_w1), 0)[1], 0)

    def drain(sem_v, sem_w):
        pltpu.make_async_copy(
            v_hbm.at[pl.ds(0, RH)], rows_v.at[pl.ds(0, RH)], sem_v).wait()
        pltpu.make_async_copy(
            w_hbm.at[pl.ds(0, RH)], w_v.at[pl.ds(0, RH)], sem_w).wait()

    lane = lax.iota(jnp.int32, LANES)

    def sample_body(s, carry):
        rbase = s * F
        r = rows_v[rbase]
        acc_s = r
        acc_q = r * r
        for i in range(1, F):
            r = rows_v[rbase + i]
            acc_s = acc_s + r
            acc_q = acc_q + r * r
        part_v[s] = acc_s * acc_s - acc_q
        return carry

    drain(sem_v0, sem_w0)
    lax.fori_loop(0, SPH, sample_body, 0)
    drain(sem_v1, sem_w1)
    lax.fori_loop(SPH, BPW, sample_body, 0)

    def block_body(b, carry):
        blk = b * LANES
        row = blk + lane
        acc = jnp.zeros((LANES,), jnp.float32)
        for e in range(LANES):
            acc = acc + plsc.load_gather(
                part_v, [row, jnp.full((LANES,), e, jnp.int32)])
        lin = jnp.zeros((LANES,), jnp.float32)
        for i in range(F):
            lin = lin + plsc.load_gather(w_v, [row * F + i])
        out_v[pl.ds(blk, LANES)] = 0.5 * acc + lin
        return carry

    lax.fori_loop(0, BPW // LANES, block_body, 0)

    pltpu.sync_copy(out_v, out_hbm.at[pl.ds(wid * BPW, BPW)])


def kernel(x, linear_w, linear_b, v):
    v_rm = _transpose(v.T)
    out = _fm_sc(x.reshape(B * F), linear_w, v_rm)
    return out.reshape(B, 1) + linear_b


# speedup vs baseline: 2.3641x; 2.3641x over previous
import functools

import jax
import jax.numpy as jnp
from jax import lax
from jax.experimental import pallas as pl
from jax.experimental.pallas import tpu as pltpu
from jax.experimental.pallas import tpu_sc as plsc

B = 4096
F = 26
EMB = 16
TOTAL = 2600000
NC = 2
NS = 16
NW = NC * NS
BPW = B // NW
RPW = BPW * F
CH = 128
NCH = RPW // CH
HALF = NCH // 2
RH = HALF * CH
SPH = RH // F
LANES = 16

BT = 4096
GRID_T = (TOTAL + BT - 1) // BT

_mesh = plsc.VectorSubcoreMesh(core_axis_name="c", subcore_axis_name="s")


def _tp_body(vt_ref, out_ref):
    lane = lax.broadcasted_iota(jnp.int32, (EMB, 8 * EMB), 1)
    row = lax.broadcasted_iota(jnp.int32, (EMB, 8 * EMB), 0)
    eye128 = jnp.float32(1.0) * (lax.rem(lane, EMB) == row)
    t128 = lax.dot_general(
        vt_ref[...], eye128, (((0,), (0,)), ((), ())),
        preferred_element_type=jnp.float32)
    t3 = t128.reshape(BT // 8, 8, 8 * EMB)
    acc = jnp.zeros((BT // 8, 8 * EMB), jnp.float32)
    lane2 = lax.broadcasted_iota(jnp.int32, (BT // 8, 8 * EMB), 1)
    for s in range(8):
        acc = jnp.where(lane2 // EMB == s, t3[:, s, :], acc)
    out_ref[...] = acc


_transpose = pl.pallas_call(
    _tp_body,
    grid=(GRID_T,),
    in_specs=[pl.BlockSpec((EMB, BT), lambda g: (0, g))],
    out_specs=pl.BlockSpec((BT // 8, 8 * EMB), lambda g: (g, 0)),
    out_shape=jax.ShapeDtypeStruct((TOTAL // 8, 8 * EMB), jnp.float32),
)


@functools.partial(
    pl.kernel,
    out_type=jax.ShapeDtypeStruct((B,), jnp.float32),
    mesh=_mesh,
    compiler_params=pltpu.CompilerParams(
        needs_layout_passes=False, use_tc_tiling_on_sc=False),
    scratch_types=[
        pltpu.VMEM((RPW,), jnp.int32),
        pltpu.VMEM((RPW, EMB), jnp.float32),
        pltpu.VMEM((RPW,), jnp.float32),
        pltpu.VMEM((BPW, LANES), jnp.float32),
        pltpu.VMEM((BPW,), jnp.float32),
        pltpu.SemaphoreType.DMA,
        pltpu.SemaphoreType.DMA,
        pltpu.SemaphoreType.DMA,
        pltpu.SemaphoreType.DMA,
    ],
)
def _fm_sc(x_hbm, w_hbm, v_hbm, out_hbm,
           idx_v, rows_v, w_v, part_v, out_v, sem_v0, sem_v1, sem_w0, sem_w1):
    wid = lax.axis_index("s") * NC + lax.axis_index("c")
    base = wid * RPW

    pltpu.sync_copy(x_hbm.at[pl.ds(base, RPW)], idx_v)

    def fire(j, sem_v, sem_w):
        off = j * CH
        idx_c = idx_v.at[pl.ds(off, CH)]
        pltpu.async_copy(v_hbm.at[idx_c], rows_v.at[pl.ds(off, CH)], sem_v)
        pltpu.async_copy(w_hbm.at[idx_c], w_v.at[pl.ds(off, CH)], sem_w)

    lax.fori_loop(0, HALF, lambda j, _: (fire(j, sem_v0, sem_w0), 0)[1], 0)
    lax.fori_loop(HALF, NCH, lambda j, _: (fire(j, sem_v1, sem_w1), 0)[1], 0)

    def drain(sem_v, sem_w):
        pltpu.make_async_copy(
            v_hbm.at[pl.ds(0, RH)], rows_v.at[pl.ds(0, RH)], sem_v).wait()
        pltpu.make_async_copy(
            w_hbm.at[pl.ds(0, RH)], w_v.at[pl.ds(0, RH)], sem_w).wait()

    lane = lax.iota(jnp.int32, LANES)

    def sample_body(s, carry):
        rbase = s * F
        r = rows_v[rbase]
        acc_s = r
        acc_q = r * r
        for i in range(1, F):
            r = rows_v[rbase + i]
            acc_s = acc_s + r
            acc_q = acc_q + r * r
        part_v[s] = acc_s * acc_s - acc_q
        return carry

    drain(sem_v0, sem_w0)
    lax.fori_loop(0, SPH, sample_body, 0)
    drain(sem_v1, sem_w1)
    lax.fori_loop(SPH, BPW, sample_body, 0)

    def block_body(b, carry):
        blk = b * LANES
        row = blk + lane
        acc = jnp.zeros((LANES,), jnp.float32)
        for e in range(LANES):
            acc = acc + plsc.load_gather(
                part_v, [row, jnp.full((LANES,), e, jnp.int32)])
        lin = jnp.zeros((LANES,), jnp.float32)
        for i in range(F):
            lin = lin + plsc.load_gather(w_v, [row * F + i])
        out_v[pl.ds(blk, LANES)] = 0.5 * acc + lin
        return carry

    lax.fori_loop(0, BPW // LANES, block_body, 0)

    pltpu.sync_copy(out_v, out_hbm.at[pl.ds(wid * BPW, BPW)])


def kernel(x, linear_w, linear_b, v):
    v_rm = _transpose(v.T)
    out = _fm_sc(x.reshape(B * F), linear_w, v_rm.reshape(TOTAL, EMB))
    return out.reshape(B, 1) + linear_b


# speedup vs baseline: 2.5957x; 1.0980x over previous
import functools

import jax
import jax.numpy as jnp
from jax import lax
from jax.experimental import pallas as pl
from jax.experimental.pallas import tpu as pltpu
from jax.experimental.pallas import tpu_sc as plsc

B = 4096
F = 26
EMB = 16
TOTAL = 2600000
NC = 2
NS = 16
NW = NC * NS
BPW = B // NW
RPW = BPW * F
CH = 128
NCH = RPW // CH
HALF = NCH // 2
RH = HALF * CH
SPH = RH // F
LANES = 16

BT = 8192
GRID_T = (TOTAL + BT - 1) // BT

_mesh = plsc.VectorSubcoreMesh(core_axis_name="c", subcore_axis_name="s")


def _tp_body(vt_ref, out_ref):
    lane = lax.broadcasted_iota(jnp.int32, (EMB, 8 * EMB), 1)
    row = lax.broadcasted_iota(jnp.int32, (EMB, 8 * EMB), 0)
    eye128 = jnp.float32(1.0) * (lax.rem(lane, EMB) == row)
    t128 = lax.dot_general(
        vt_ref[...], eye128, (((0,), (0,)), ((), ())),
        preferred_element_type=jnp.float32)
    t3 = t128.reshape(BT // 8, 8, 8 * EMB)
    acc = jnp.zeros((BT // 8, 8 * EMB), jnp.float32)
    lane2 = lax.broadcasted_iota(jnp.int32, (BT // 8, 8 * EMB), 1)
    for s in range(8):
        acc = jnp.where(lane2 // EMB == s, t3[:, s, :], acc)
    out_ref[...] = acc


_transpose = pl.pallas_call(
    _tp_body,
    grid=(GRID_T,),
    in_specs=[pl.BlockSpec((EMB, BT), lambda g: (0, g))],
    out_specs=pl.BlockSpec((BT // 8, 8 * EMB), lambda g: (g, 0)),
    out_shape=jax.ShapeDtypeStruct((TOTAL // 8, 8 * EMB), jnp.float32),
)


@functools.partial(
    pl.kernel,
    out_type=jax.ShapeDtypeStruct((B,), jnp.float32),
    mesh=_mesh,
    compiler_params=pltpu.CompilerParams(
        needs_layout_passes=False, use_tc_tiling_on_sc=False),
    scratch_types=[
        pltpu.VMEM((RPW,), jnp.int32),
        pltpu.VMEM((RPW, EMB), jnp.float32),
        pltpu.VMEM((RPW,), jnp.float32),
        pltpu.VMEM((BPW, LANES), jnp.float32),
        pltpu.VMEM((BPW,), jnp.float32),
        pltpu.SemaphoreType.DMA,
        pltpu.SemaphoreType.DMA,
        pltpu.SemaphoreType.DMA,
        pltpu.SemaphoreType.DMA,
    ],
)
def _fm_sc(x_hbm, w_hbm, v_hbm, out_hbm,
           idx_v, rows_v, w_v, part_v, out_v, sem_v0, sem_v1, sem_w0, sem_w1):
    wid = lax.axis_index("s") * NC + lax.axis_index("c")
    base = wid * RPW

    pltpu.sync_copy(x_hbm.at[pl.ds(base, RPW)], idx_v)

    def fire(j, sem_v, sem_w):
        off = j * CH
        idx_c = idx_v.at[pl.ds(off, CH)]
        pltpu.async_copy(v_hbm.at[idx_c], rows_v.at[pl.ds(off, CH)], sem_v)
        pltpu.async_copy(w_hbm.at[idx_c], w_v.at[pl.ds(off, CH)], sem_w)

    lax.fori_loop(0, HALF, lambda j, _: (fire(j, sem_v0, sem_w0), 0)[1], 0)
    lax.fori_loop(HALF, NCH, lambda j, _: (fire(j, sem_v1, sem_w1), 0)[1], 0)

    def drain(sem_v, sem_w):
        pltpu.make_async_copy(
            v_hbm.at[pl.ds(0, RH)], rows_v.at[pl.ds(0, RH)], sem_v).wait()
        pltpu.make_async_copy(
            w_hbm.at[pl.ds(0, RH)], w_v.at[pl.ds(0, RH)], sem_w).wait()

    lane = lax.iota(jnp.int32, LANES)

    def sample_body(s, carry):
        rbase = s * F
        r = rows_v[rbase]
        acc_s = r
        acc_q = r * r
        for i in range(1, F):
            r = rows_v[rbase + i]
            acc_s = acc_s + r
            acc_q = acc_q + r * r
        part_v[s] = acc_s * acc_s - acc_q
        return carry

    drain(sem_v0, sem_w0)
    lax.fori_loop(0, SPH, sample_body, 0)
    drain(sem_v1, sem_w1)
    lax.fori_loop(SPH, BPW, sample_body, 0)

    def block_body(b, carry):
        blk = b * LANES
        row = blk + lane
        acc = jnp.zeros((LANES,), jnp.float32)
        for e in range(LANES):
            acc = acc + plsc.load_gather(
                part_v, [row, jnp.full((LANES,), e, jnp.int32)])
        lin = jnp.zeros((LANES,), jnp.float32)
        for i in range(F):
            lin = lin + plsc.load_gather(w_v, [row * F + i])
        out_v[pl.ds(blk, LANES)] = 0.5 * acc + lin
        return carry

    lax.fori_loop(0, BPW // LANES, block_body, 0)

    pltpu.sync_copy(out_v, out_hbm.at[pl.ds(wid * BPW, BPW)])


def kernel(x, linear_w, linear_b, v):
    v_rm = _transpose(v.T)
    out = _fm_sc(x.reshape(B * F), linear_w, v_rm.reshape(TOTAL, EMB))
    return out.reshape(B, 1) + linear_b
